# baseline (speedup 1.0000x reference)
import functools

import jax
import jax.numpy as jnp
from jax import lax
from jax.experimental import pallas as pl
from jax.experimental.pallas import tpu as pltpu
from jax.experimental.pallas import tpu_sc as plsc

_B, _H, _D = 16384, 20, 64
_N = _B * _H
_NC, _NS = 2, 16
_NW = _NC * _NS
_PER_W = _N // _NW
_CHUNK = 128
_NCHUNK = _PER_W // _CHUNK


def _body(idx_hbm, table_hbm, out_hbm, idx_v, rows_v, gsem):
    wid = lax.axis_index("s") * _NC + lax.axis_index("c")
    pltpu.sync_copy(idx_hbm.at[wid], idx_v)
    base = wid * _PER_W

    def step(j, carry):
        pltpu.async_copy(table_hbm.at[idx_v.at[j]], rows_v, gsem).wait()
        pltpu.sync_copy(rows_v, out_hbm.at[pl.ds(base + j * _CHUNK, _CHUNK)])
        return carry

    lax.fori_loop(0, _NCHUNK, step, 0)


_gather = functools.partial(
    pl.kernel,
    out_type=jax.ShapeDtypeStruct((_N, _D), jnp.float32),
    mesh=plsc.VectorSubcoreMesh(core_axis_name="c", subcore_axis_name="s"),
    compiler_params=pltpu.CompilerParams(use_tc_tiling_on_sc=False),
    scratch_types=[
        pltpu.VMEM((_NCHUNK, _CHUNK), jnp.int32),
        pltpu.VMEM((_CHUNK, _D), jnp.float32),
        pltpu.SemaphoreType.DMA,
    ],
)(_body)


def kernel(indices, weight):
    flat = indices.astype(jnp.int32).reshape(_NW, _NCHUNK, _CHUNK)
    out = _gather(flat, weight)
    return out.reshape(_B, _H, _D)


# speedup vs baseline: 1.0667x; 1.0667x over previous
import functools

import jax
import jax.numpy as jnp
from jax import lax
from jax.experimental import pallas as pl
from jax.experimental.pallas import tpu as pltpu
from jax.experimental.pallas import tpu_sc as plsc

_B, _H, _D = 16384, 20, 64
_N = _B * _H
_NC, _NS = 2, 16
_NW = _NC * _NS
_PER_W = _N // _NW
_CHUNK = 128
_NCHUNK = _PER_W // _CHUNK
_NBUF = 8
_LOOK = 4


def _body(idx_hbm, table_hbm, out_hbm, idx_v, rows_v, gsem, ssem):
    wid = lax.axis_index("s") * _NC + lax.axis_index("c")
    pltpu.sync_copy(idx_hbm.at[wid], idx_v)
    base = wid * _PER_W

    def start_gather(j, buf):
        pltpu.async_copy(table_hbm.at[idx_v.at[j]], rows_v.at[buf], gsem)

    def wait_gather(j, buf):
        pltpu.make_async_copy(
            table_hbm.at[idx_v.at[j]], rows_v.at[buf], gsem).wait()

    def start_store(j, buf):
        pltpu.async_copy(
            rows_v.at[buf], out_hbm.at[pl.ds(base + j * _CHUNK, _CHUNK)], ssem)

    def wait_store(j, buf):
        pltpu.make_async_copy(
            rows_v.at[buf], out_hbm.at[pl.ds(base + j * _CHUNK, _CHUNK)],
            ssem).wait()

    for j in range(_LOOK):
        start_gather(j, j)

    for j in range(_NBUF - _LOOK):
        wait_gather(j, j)
        start_store(j, j)
        start_gather(j + _LOOK, j + _LOOK)

    def step(j, carry):
        buf = lax.rem(j, _NBUF)
        wait_gather(j, buf)
        start_store(j, buf)
        nxt = j + _LOOK
        nbuf = lax.rem(nxt, _NBUF)
        wait_store(nxt - _NBUF, nbuf)
        start_gather(nxt, nbuf)
        return carry

    lax.fori_loop(_NBUF - _LOOK, _NCHUNK - _LOOK, step, 0)

    for j in range(_NCHUNK - _LOOK, _NCHUNK):
        buf = j % _NBUF
        wait_gather(j, buf)
        start_store(j, buf)
    for j in range(_NCHUNK - _NBUF, _NCHUNK):
        wait_store(j, j % _NBUF)


_gather = functools.partial(
    pl.kernel,
    out_type=jax.ShapeDtypeStruct((_N, _D), jnp.float32),
    mesh=plsc.VectorSubcoreMesh(core_axis_name="c", subcore_axis_name="s"),
    compiler_params=pltpu.CompilerParams(use_tc_tiling_on_sc=False),
    scratch_types=[
        pltpu.VMEM((_NCHUNK, _CHUNK), jnp.int32),
        pltpu.VMEM((_NBUF, _CHUNK, _D), jnp.float32),
        pltpu.SemaphoreType.DMA,
        pltpu.SemaphoreType.DMA,
    ],
)(_body)


def kernel(indices, weight):
    flat = indices.astype(jnp.int32).reshape(_NW, _NCHUNK, _CHUNK)
    out = _gather(flat, weight)
    return out.reshape(_B, _H, _D)
